# Initial kernel scaffold; baseline (speedup 1.0000x reference)
import jax
import jax.numpy as jnp
from jax.experimental import pallas as pl


def kernel(packed_inputs, valid_load, w1, b1, w2, b2):
    raise NotImplementedError("write your pallas kernel here")


# speedup vs baseline: 2.4235x; 2.4235x over previous
import jax
import jax.numpy as jnp
from jax.experimental import pallas as pl
from jax.experimental.pallas import tpu as pltpu

_E, _CAP, _D = 64, 1024, 512
_H = 4 * _D
_CB = 256


def _ffn_body(vl_ref, x_ref, w1_ref, b1_ref, w2_ref, b2_ref, o_ref):
    e = pl.program_id(0)
    c = pl.program_id(1)
    valid = vl_ref[e]
    base = c * _CB

    @pl.when(base < valid)
    def _compute():
        x = x_ref[0]
        h = jnp.dot(x, w1_ref[0], preferred_element_type=jnp.float32)
        h = h + b1_ref[0]
        h = 0.5 * h * (1.0 + jax.lax.erf(h * 0.7071067811865476))
        y = jnp.dot(h, w2_ref[0], preferred_element_type=jnp.float32)
        y = y + b2_ref[0]
        rows = base + jax.lax.broadcasted_iota(jnp.int32, (_CB, 1), 0)
        o_ref[0] = jnp.where(rows < valid, y, 0.0)

    @pl.when(base >= valid)
    def _zero():
        o_ref[...] = jnp.zeros_like(o_ref)


def kernel(packed_inputs, valid_load, w1, b1, w2, b2):
    vl = valid_load.astype(jnp.int32)
    b1r = b1.reshape(_E, 1, _H)
    b2r = b2.reshape(_E, 1, _D)

    grid = (_E, _CAP // _CB)
    out = pl.pallas_call(
        _ffn_body,
        out_shape=jax.ShapeDtypeStruct((_E, _CAP, _D), jnp.float32),
        grid_spec=pltpu.PrefetchScalarGridSpec(
            num_scalar_prefetch=1,
            grid=grid,
            in_specs=[
                pl.BlockSpec((1, _CB, _D), lambda e, c, vl_ref: (e, c, 0)),
                pl.BlockSpec((1, _D, _H), lambda e, c, vl_ref: (e, 0, 0)),
                pl.BlockSpec((1, 1, _H), lambda e, c, vl_ref: (e, 0, 0)),
                pl.BlockSpec((1, _H, _D), lambda e, c, vl_ref: (e, 0, 0)),
                pl.BlockSpec((1, 1, _D), lambda e, c, vl_ref: (e, 0, 0)),
            ],
            out_specs=pl.BlockSpec((1, _CB, _D), lambda e, c, vl_ref: (e, c, 0)),
        ),
        compiler_params=pltpu.CompilerParams(
            dimension_semantics=("parallel", "arbitrary"),
            vmem_limit_bytes=56 * 1024 * 1024,
        ),
        name="fused_group_ffn",
    )(vl, packed_inputs, w1, b1r, w2, b2r)
    return out


# speedup vs baseline: 2.4856x; 1.0256x over previous
import jax
import jax.numpy as jnp
from jax.experimental import pallas as pl
from jax.experimental.pallas import tpu as pltpu

_E, _CAP, _D = 64, 1024, 512
_H = 4 * _D
_CB = 256


def _ffn_body(vl_ref, x_ref, w1_ref, b1_ref, w2_ref, b2_ref, o_ref):
    e = pl.program_id(0)
    c = pl.program_id(1)
    valid = vl_ref[e]
    base = c * _CB

    @pl.when(base < valid)
    def _compute():
        x = x_ref[0]
        h = jnp.dot(x, w1_ref[0], preferred_element_type=jnp.float32)
        h = h + b1_ref[0]
        h = 0.5 * h * (1.0 + jax.lax.erf(h * 0.7071067811865476))
        y = jnp.dot(h, w2_ref[0], preferred_element_type=jnp.float32)
        y = y + b2_ref[0]
        rows = base + jax.lax.broadcasted_iota(jnp.int32, (_CB, 1), 0)
        o_ref[0] = jnp.where(rows < valid, y, 0.0)

    @pl.when(base >= valid)
    def _zero():
        o_ref[...] = jnp.zeros_like(o_ref)


def kernel(packed_inputs, valid_load, w1, b1, w2, b2):
    vl = valid_load.astype(jnp.int32)
    b1r = b1.reshape(_E, 1, _H)
    b2r = b2.reshape(_E, 1, _D)

    grid = (_E, _CAP // _CB)

    def _xmap(e, c, vl_ref):
        last_valid = jnp.maximum((vl_ref[e] + _CB - 1) // _CB - 1, 0)
        return (e, jnp.minimum(c, last_valid), 0)

    def _emap(e, c, vl_ref):
        return (e, 0, 0)

    out = pl.pallas_call(
        _ffn_body,
        out_shape=jax.ShapeDtypeStruct((_E, _CAP, _D), jnp.float32),
        grid_spec=pltpu.PrefetchScalarGridSpec(
            num_scalar_prefetch=1,
            grid=grid,
            in_specs=[
                pl.BlockSpec((1, _CB, _D), _xmap),
                pl.BlockSpec((1, _D, _H), _emap),
                pl.BlockSpec((1, 1, _H), _emap),
                pl.BlockSpec((1, _H, _D), _emap),
                pl.BlockSpec((1, 1, _D), _emap),
            ],
            out_specs=pl.BlockSpec((1, _CB, _D), lambda e, c, vl_ref: (e, c, 0)),
        ),
        compiler_params=pltpu.CompilerParams(
            dimension_semantics=("parallel", "arbitrary"),
            vmem_limit_bytes=56 * 1024 * 1024,
        ),
        name="fused_group_ffn",
    )(vl, packed_inputs, w1, b1r, w2, b2r)
    return out
